# baseline (speedup 1.0000x reference)
import functools

import jax
import jax.numpy as jnp
from jax import lax
from jax.experimental import pallas as pl
from jax.experimental.pallas import tpu as pltpu
from jax.experimental.pallas import tpu_sc as plsc

DIM = 32
WORDS = DIM // 4
NC, NS, L = 2, 16, 16
NW = NC * NS
CHUNK = 128


def _sc_lookup(x3d, table, scales):
    nw, rows_per_w, chunk = x3d.shape
    assert chunk == CHUNK and nw == NW
    tot = nw * rows_per_w * chunk
    per_w = tot // NW
    n_chunks = per_w // CHUNK
    n_pairs = n_chunks // 2

    mesh = plsc.VectorSubcoreMesh(core_axis_name="c", subcore_axis_name="s")

    @functools.partial(
        pl.kernel,
        mesh=mesh,
        compiler_params=pltpu.CompilerParams(
            use_tc_tiling_on_sc=False, needs_layout_passes=False),
        out_type=jax.ShapeDtypeStruct((tot, DIM), jnp.float32),
        scratch_types=[
            pltpu.VMEM((rows_per_w, CHUNK), jnp.int32),
            pltpu.VMEM((CHUNK, WORDS), jnp.int32),
            pltpu.VMEM((CHUNK, WORDS), jnp.int32),
            pltpu.VMEM((CHUNK,), jnp.float32),
            pltpu.VMEM((CHUNK,), jnp.float32),
            pltpu.VMEM((CHUNK, DIM), jnp.float32),
            pltpu.VMEM((CHUNK, DIM), jnp.float32),
            pltpu.SemaphoreType.DMA,
            pltpu.SemaphoreType.DMA,
            pltpu.SemaphoreType.DMA,
            pltpu.SemaphoreType.DMA,
            pltpu.SemaphoreType.DMA,
            pltpu.SemaphoreType.DMA,
        ],
    )
    def k(x_hbm, tab_hbm, scl_hbm, out_hbm, idx_v, rows0, rows1,
          scl0, scl1, ob0, ob1, gsem0, gsem1, ssem0, ssem1, osem0, osem1):
        wid = lax.axis_index("s") * NC + lax.axis_index("c")
        out_base = wid * per_w

        pltpu.sync_copy(x_hbm.at[wid], idx_v)

        def issue_gather(c, rows_buf, scl_buf, gsem, ssem):
            pltpu.async_copy(tab_hbm.at[idx_v.at[c]], rows_buf, gsem)
            pltpu.async_copy(scl_hbm.at[idx_v.at[c]], scl_buf, ssem)

        def wait_gather(c, rows_buf, scl_buf, gsem, ssem):
            pltpu.make_async_copy(tab_hbm.at[idx_v.at[c]], rows_buf, gsem).wait()
            pltpu.make_async_copy(scl_hbm.at[idx_v.at[c]], scl_buf, ssem).wait()

        def issue_out(c, ob, osem):
            pltpu.async_copy(
                ob, out_hbm.at[pl.ds(out_base + c * CHUNK, CHUNK)], osem)

        def wait_out(ob, osem):
            pltpu.make_async_copy(
                ob, out_hbm.at[pl.ds(out_base, CHUNK)], osem).wait()

        riota = lax.iota(jnp.int32, L)

        def compute(rows_buf, scl_buf, ob):
            def grp(g, carry):
                rb = g * L
                scl = scl_buf[pl.ds(rb, L)]
                row_idx = riota + rb
                for j in range(WORDS):
                    jv = jnp.full((L,), j, jnp.int32)
                    w = plsc.load_gather(rows_buf, [row_idx, jv])
                    for kk in range(4):
                        col = 4 * j + kk
                        if kk < 3:
                            v = jnp.right_shift(jnp.left_shift(w, 24 - 8 * kk), 24)
                        else:
                            v = jnp.right_shift(w, 24)
                        f = v.astype(jnp.float32) * scl
                        cv = jnp.full((L,), col, jnp.int32)
                        plsc.store_scatter(ob, [row_idx, cv], f)
                return carry
            lax.fori_loop(0, CHUNK // L, grp, 0)

        issue_gather(0, rows0, scl0, gsem0, ssem0)
        issue_gather(1, rows1, scl1, gsem1, ssem1)

        def body(i, carry):
            a = 2 * i
            b = a + 1

            wait_gather(a, rows0, scl0, gsem0, ssem0)

            @pl.when(i > 0)
            def _w0():
                wait_out(ob0, osem0)

            compute(rows0, scl0, ob0)
            issue_out(a, ob0, osem0)

            @pl.when(i < n_pairs - 1)
            def _p0():
                issue_gather(a + 2, rows0, scl0, gsem0, ssem0)

            wait_gather(b, rows1, scl1, gsem1, ssem1)

            @pl.when(i > 0)
            def _w1():
                wait_out(ob1, osem1)

            compute(rows1, scl1, ob1)
            issue_out(b, ob1, osem1)

            @pl.when(i < n_pairs - 1)
            def _p1():
                issue_gather(b + 2, rows1, scl1, gsem1, ssem1)

            return carry

        lax.fori_loop(0, n_pairs, body, 0)
        wait_out(ob0, osem0)
        wait_out(ob1, osem1)

    return k(x3d, table, scales)


def kernel(x, weight_q, scales):
    batch, hist = x.shape
    num_emb = weight_q.shape[0]
    x3d = x.reshape(NW, -1, CHUNK)
    table = lax.bitcast_convert_type(
        weight_q.reshape(num_emb, WORDS, 4), jnp.int32)
    out = _sc_lookup(x3d, table, scales)
    return out.reshape(batch, hist, DIM)


# speedup vs baseline: 1.1243x; 1.1243x over previous
import functools

import jax
import jax.numpy as jnp
from jax import lax
from jax.experimental import pallas as pl
from jax.experimental.pallas import tpu as pltpu
from jax.experimental.pallas import tpu_sc as plsc

DIM = 32
WORDS = DIM // 4
NC, NS, L = 2, 16, 16
NW = NC * NS
BROWS = 8
NBUF = 4


def _sc_lookup(x3d, table, scales, batch, hist):
    nw, n_chunks, cr = x3d.shape
    rows_per_w = batch // NW
    n_outer = n_chunks // NBUF
    n_grp = cr // L

    mesh = plsc.VectorSubcoreMesh(core_axis_name="c", subcore_axis_name="s")

    scratch = [pltpu.VMEM((n_chunks, cr), jnp.int32)]
    scratch += [pltpu.VMEM((cr, WORDS), jnp.int32)
                for _ in range(NBUF)]
    scratch += [pltpu.VMEM((cr,), jnp.float32)
                for _ in range(NBUF)]
    scratch += [pltpu.VMEM((BROWS, hist, DIM), jnp.float32)
                for _ in range(NBUF)]
    scratch += [pltpu.SemaphoreType.DMA for _ in range(3 * NBUF)]

    @functools.partial(
        pl.kernel,
        mesh=mesh,
        compiler_params=pltpu.CompilerParams(
            use_tc_tiling_on_sc=False, needs_layout_passes=False),
        out_type=jax.ShapeDtypeStruct((batch, hist, DIM), jnp.float32),
        scratch_types=scratch,
    )
    def k(x_hbm, tab_hbm, scl_hbm, out_hbm, idx_v, *bufs):
        rows = bufs[0:NBUF]
        scl = bufs[NBUF:2 * NBUF]
        ob = bufs[2 * NBUF:3 * NBUF]
        gsem = bufs[3 * NBUF:4 * NBUF]
        ssem = bufs[4 * NBUF:5 * NBUF]
        osem = bufs[5 * NBUF:6 * NBUF]

        wid = lax.axis_index("s") * NC + lax.axis_index("c")
        row0 = wid * rows_per_w

        pltpu.sync_copy(x_hbm.at[wid], idx_v)

        def issue_gather(c, b):
            idx = idx_v.at[c]
            pltpu.async_copy(tab_hbm.at[idx], rows[b], gsem[b])
            pltpu.async_copy(scl_hbm.at[idx], scl[b], ssem[b])

        def wait_gather(c, b):
            idx = idx_v.at[c]
            pltpu.make_async_copy(tab_hbm.at[idx], rows[b], gsem[b]).wait()
            pltpu.make_async_copy(scl_hbm.at[idx], scl[b], ssem[b]).wait()

        def issue_out(c, b):
            pltpu.async_copy(
                ob[b], out_hbm.at[pl.ds(row0 + c * BROWS, BROWS)], osem[b])

        def wait_out(b):
            pltpu.make_async_copy(
                ob[b], out_hbm.at[pl.ds(row0, BROWS)], osem[b]).wait()

        riota = lax.iota(jnp.int32, L)
        hist_i = jnp.int32(hist)

        def compute(b):
            rows_b, scl_b, ob_b = rows[b], scl[b], ob[b]

            def grp(g, carry):
                r = riota + g * L
                d = r // hist_i
                m = r - d * hist_i
                s = plsc.load_gather(scl_b, [r])
                for j in range(WORDS):
                    jv = jnp.full((L,), j, jnp.int32)
                    w = plsc.load_gather(rows_b, [r, jv])
                    for kk in range(4):
                        col = 4 * j + kk
                        if kk < 3:
                            v = jnp.right_shift(jnp.left_shift(w, 24 - 8 * kk), 24)
                        else:
                            v = jnp.right_shift(w, 24)
                        f = v.astype(jnp.float32) * s
                        cv = jnp.full((L,), col, jnp.int32)
                        plsc.store_scatter(ob_b, [d, m, cv], f)
                return carry
            lax.fori_loop(0, n_grp, grp, 0)

        for b in range(NBUF):
            issue_gather(b, b)

        def body(i, carry):
            for b in range(NBUF):
                c = i * NBUF + b
                wait_gather(c, b)

                @pl.when(i > 0)
                def _wo():
                    wait_out(b)

                compute(b)
                issue_out(c, b)

                @pl.when(i < n_outer - 1)
                def _pg():
                    issue_gather(c + NBUF, b)

            return carry

        lax.fori_loop(0, n_outer, body, 0)
        for b in range(NBUF):
            wait_out(b)

    return k(x3d, table, scales)


def kernel(x, weight_q, scales):
    batch, hist = x.shape
    num_emb = weight_q.shape[0]
    cr = BROWS * hist
    x3d = x.reshape(NW, -1, cr)
    table = lax.bitcast_convert_type(
        weight_q.reshape(num_emb, WORDS, 4), jnp.int32)
    return _sc_lookup(x3d, table, scales, batch, hist)


# speedup vs baseline: 1.6775x; 1.4920x over previous
import functools

import jax
import jax.numpy as jnp
from jax import lax
from jax.experimental import pallas as pl
from jax.experimental.pallas import tpu as pltpu
from jax.experimental.pallas import tpu_sc as plsc

DIM = 32
WORDS = DIM // 4
NC, NS, L = 2, 16, 16
NW = NC * NS
BROWS = 8
NBUF = 4
OBUF = 2
PCH = 1250

_SC_PARAMS = pltpu.CompilerParams(
    use_tc_tiling_on_sc=False, needs_layout_passes=False)


def _sc_prep(wq_flat, num_emb):
    rows_per_w = num_emb // NW
    n_chunks = rows_per_w // PCH
    n_iter = PCH * DIM // 64

    mesh = plsc.VectorSubcoreMesh(core_axis_name="c", subcore_axis_name="s")

    scratch = [pltpu.VMEM((PCH * DIM,), jnp.int8) for _ in range(2)]
    scratch += [pltpu.VMEM((PCH, WORDS), jnp.int32) for _ in range(2)]
    scratch += [pltpu.SemaphoreType.DMA for _ in range(4)]

    @functools.partial(
        pl.kernel,
        mesh=mesh,
        compiler_params=_SC_PARAMS,
        out_type=jax.ShapeDtypeStruct((num_emb, WORDS), jnp.int32),
        scratch_types=scratch,
    )
    def k(wq_hbm, out_hbm, ib0, ib1, ob0, ob1, is0, is1, os0, os1):
        ib = (ib0, ib1)
        ob = (ob0, ob1)
        isem = (is0, is1)
        osem = (os0, os1)
        wid = lax.axis_index("s") * NC + lax.axis_index("c")
        byte0 = wid * rows_per_w * DIM
        row0 = wid * rows_per_w

        liota = lax.iota(jnp.int32, L)
        dstat = jnp.right_shift(liota, 3)
        mstat = jnp.bitwise_and(liota, 7)

        def issue_in(c, p):
            pltpu.async_copy(
                wq_hbm.at[pl.ds(byte0 + c * PCH * DIM, PCH * DIM)],
                ib[p], isem[p])

        def wait_in(p):
            pltpu.make_async_copy(
                wq_hbm.at[pl.ds(byte0, PCH * DIM)], ib[p], isem[p]).wait()

        def issue_out(c, p):
            pltpu.async_copy(
                ob[p], out_hbm.at[pl.ds(row0 + c * PCH, PCH)], osem[p])

        def wait_out(p):
            pltpu.make_async_copy(
                ob[p], out_hbm.at[pl.ds(row0, PCH)], osem[p]).wait()

        def convert(p):
            ib_p, ob_p = ib[p], ob[p]

            def step(t, carry):
                v = ib_p[pl.ds(t * 64, 64)]
                w = plsc.bitcast(v, jnp.int32)
                plsc.store_scatter(ob_p, [dstat + t * 2, mstat], w)
                return carry
            lax.fori_loop(0, n_iter, step, 0)

        issue_in(0, 0)
        issue_in(1, 1)

        def body(i, carry):
            for p in range(2):
                c = i * 2 + p
                wait_in(p)

                @pl.when(i > 0)
                def _wo():
                    wait_out(p)

                convert(p)
                issue_out(c, p)

                @pl.when(c + 2 < n_chunks)
                def _pi():
                    issue_in(c + 2, p)

            return carry

        lax.fori_loop(0, n_chunks // 2, body, 0)
        if n_chunks % 2:
            c_last = n_chunks - 1
            wait_in(0)
            wait_out(0)
            convert(0)
            issue_out(c_last, 0)
        wait_out(0)
        wait_out(1)

    return k(wq_flat)


def _sc_lookup(x3d, table, scales, batch, hist):
    nw, n_chunks, cr = x3d.shape
    rows_per_w = batch // NW
    n_outer = n_chunks // NBUF
    n_grp = cr // L

    mesh = plsc.VectorSubcoreMesh(core_axis_name="c", subcore_axis_name="s")

    scratch = [pltpu.VMEM((n_chunks, cr), jnp.int32)]
    scratch += [pltpu.VMEM((cr, WORDS), jnp.int32)
                for _ in range(NBUF)]
    scratch += [pltpu.VMEM((cr,), jnp.float32)
                for _ in range(NBUF)]
    scratch += [pltpu.VMEM((BROWS, hist, DIM), jnp.float32)
                for _ in range(OBUF)]
    scratch += [pltpu.SemaphoreType.DMA for _ in range(2 * NBUF)]
    scratch += [pltpu.SemaphoreType.DMA for _ in range(OBUF)]

    @functools.partial(
        pl.kernel,
        mesh=mesh,
        compiler_params=_SC_PARAMS,
        out_type=jax.ShapeDtypeStruct((batch, hist, DIM), jnp.float32),
        scratch_types=scratch,
    )
    def k(x_hbm, tab_hbm, scl_hbm, out_hbm, idx_v, *bufs):
        rows = bufs[0:NBUF]
        scl = bufs[NBUF:2 * NBUF]
        ob = bufs[2 * NBUF:2 * NBUF + OBUF]
        gsem = bufs[2 * NBUF + OBUF:3 * NBUF + OBUF]
        ssem = bufs[3 * NBUF + OBUF:4 * NBUF + OBUF]
        osem = bufs[4 * NBUF + OBUF:4 * NBUF + 2 * OBUF]

        wid = lax.axis_index("s") * NC + lax.axis_index("c")
        row0 = wid * rows_per_w

        pltpu.sync_copy(x_hbm.at[wid], idx_v)

        def issue_gather(c, b):
            pltpu.async_copy(tab_hbm.at[idx_v.at[c]], rows[b], gsem[b])
            pltpu.async_copy(scl_hbm.at[idx_v.at[c]], scl[b], ssem[b])

        def wait_gather(c, b):
            pltpu.make_async_copy(tab_hbm.at[idx_v.at[c]], rows[b], gsem[b]).wait()
            pltpu.make_async_copy(scl_hbm.at[idx_v.at[c]], scl[b], ssem[b]).wait()

        def issue_out(c, o):
            pltpu.async_copy(
                ob[o], out_hbm.at[pl.ds(row0 + c * BROWS, BROWS)], osem[o])

        def wait_out(o):
            pltpu.make_async_copy(
                ob[o], out_hbm.at[pl.ds(row0, BROWS)], osem[o]).wait()

        riota = lax.iota(jnp.int32, L)
        hist_i = jnp.int32(hist)

        def compute(b, o):
            rows_b, scl_b, ob_o = rows[b], scl[b], ob[o]

            def grp(g, carry):
                rb = g * L
                r = riota + rb
                d = r // hist_i
                m = r - d * hist_i
                s = plsc.load_gather(scl_b, [r])
                for j in range(WORDS):
                    jv = jnp.full((L,), j, jnp.int32)
                    w = plsc.load_gather(rows_b, [r, jv])
                    for kk in range(4):
                        col = 4 * j + kk
                        if kk < 3:
                            v = jnp.right_shift(jnp.left_shift(w, 24 - 8 * kk), 24)
                        else:
                            v = jnp.right_shift(w, 24)
                        f = v.astype(jnp.float32) * s
                        cv = jnp.full((L,), col, jnp.int32)
                        plsc.store_scatter(ob_o, [d, m, cv], f)
                return carry
            lax.fori_loop(0, n_grp, grp, 0)

        for b in range(NBUF):
            issue_gather(b, b)

        def body(i, carry):
            for b in range(NBUF):
                c = i * NBUF + b
                o = b % OBUF
                wait_gather(c, b)

                if b >= OBUF:
                    wait_out(o)
                else:
                    @pl.when(i > 0)
                    def _wo():
                        wait_out(o)

                compute(b, o)
                issue_out(c, o)

                @pl.when(i < n_outer - 1)
                def _pg():
                    issue_gather(c + NBUF, b)

            return carry

        lax.fori_loop(0, n_outer, body, 0)
        for o in range(OBUF):
            wait_out(o)

    return k(x3d, table, scales)


def kernel(x, weight_q, scales):
    batch, hist = x.shape
    num_emb = weight_q.shape[0]
    cr = BROWS * hist
    x3d = x.reshape(NW, -1, cr)
    table = _sc_prep(weight_q.reshape(-1), num_emb)
    return _sc_lookup(x3d, table, scales, batch, hist)


# speedup vs baseline: 1.6778x; 1.0002x over previous
import functools

import jax
import jax.numpy as jnp
from jax import lax
from jax.experimental import pallas as pl
from jax.experimental.pallas import tpu as pltpu
from jax.experimental.pallas import tpu_sc as plsc

DIM = 32
WORDS = DIM // 4
NC, NS, L = 2, 16, 16
NW = NC * NS
BROWS = 8
NBUF = 4
OBUF = 2
PCH = 1250

_SC_PARAMS = pltpu.CompilerParams(
    use_tc_tiling_on_sc=False, needs_layout_passes=False)


def _sc_prep(wq_pairs, num_emb):
    rows_per_w = num_emb // NW
    n_chunks = rows_per_w // PCH
    n_iter = PCH // 2
    pch2 = PCH // 2

    mesh = plsc.VectorSubcoreMesh(core_axis_name="c", subcore_axis_name="s")

    scratch = [pltpu.VMEM((PCH // 2, 2 * DIM), jnp.int8) for _ in range(2)]
    scratch += [pltpu.VMEM((PCH, WORDS), jnp.int32) for _ in range(2)]
    scratch += [pltpu.SemaphoreType.DMA for _ in range(4)]

    @functools.partial(
        pl.kernel,
        mesh=mesh,
        compiler_params=_SC_PARAMS,
        out_type=jax.ShapeDtypeStruct((num_emb, WORDS), jnp.int32),
        scratch_types=scratch,
    )
    def k(wq_hbm, out_hbm, ib0, ib1, ob0, ob1, is0, is1, os0, os1):
        ib = (ib0, ib1)
        ob = (ob0, ob1)
        isem = (is0, is1)
        osem = (os0, os1)
        wid = lax.axis_index("s") * NC + lax.axis_index("c")
        pair0 = wid * rows_per_w // 2
        row0 = wid * rows_per_w

        liota = lax.iota(jnp.int32, L)
        dstat = jnp.right_shift(liota, 3)
        mstat = jnp.bitwise_and(liota, 7)

        def issue_in(c, p):
            pltpu.async_copy(
                wq_hbm.at[pl.ds(pair0 + c * pch2, pch2)], ib[p], isem[p])

        def wait_in(p):
            pltpu.make_async_copy(
                wq_hbm.at[pl.ds(pair0, pch2)], ib[p], isem[p]).wait()

        def issue_out(c, p):
            pltpu.async_copy(
                ob[p], out_hbm.at[pl.ds(row0 + c * PCH, PCH)], osem[p])

        def wait_out(p):
            pltpu.make_async_copy(
                ob[p], out_hbm.at[pl.ds(row0, PCH)], osem[p]).wait()

        def convert(p):
            ib_p, ob_p = ib[p], ob[p]

            def step(t, carry):
                v = ib_p[t, :]
                w = plsc.bitcast(v, jnp.int32)
                plsc.store_scatter(ob_p, [dstat + t * 2, mstat], w)
                return carry
            lax.fori_loop(0, n_iter, step, 0)

        issue_in(0, 0)
        issue_in(1, 1)

        def body(i, carry):
            for p in range(2):
                c = i * 2 + p
                wait_in(p)

                @pl.when(i > 0)
                def _wo():
                    wait_out(p)

                convert(p)
                issue_out(c, p)

                @pl.when(c + 2 < n_chunks)
                def _pi():
                    issue_in(c + 2, p)

            return carry

        lax.fori_loop(0, n_chunks // 2, body, 0)
        if n_chunks % 2:
            c_last = n_chunks - 1
            wait_in(0)
            wait_out(0)
            convert(0)
            issue_out(c_last, 0)
        wait_out(0)
        wait_out(1)

    return k(wq_pairs)


def _sc_lookup(x3d, table, scales, batch, hist):
    nw, n_chunks, cr = x3d.shape
    rows_per_w = batch // NW
    n_outer = n_chunks // NBUF
    n_grp = cr // L

    mesh = plsc.VectorSubcoreMesh(core_axis_name="c", subcore_axis_name="s")

    scratch = [pltpu.VMEM((n_chunks, cr), jnp.int32)]
    scratch += [pltpu.VMEM((cr, WORDS), jnp.int32)
                for _ in range(NBUF)]
    scratch += [pltpu.VMEM((cr,), jnp.float32)
                for _ in range(NBUF)]
    scratch += [pltpu.VMEM((BROWS, hist, DIM), jnp.float32)
                for _ in range(OBUF)]
    scratch += [pltpu.SemaphoreType.DMA for _ in range(2 * NBUF)]
    scratch += [pltpu.SemaphoreType.DMA for _ in range(OBUF)]

    @functools.partial(
        pl.kernel,
        mesh=mesh,
        compiler_params=_SC_PARAMS,
        out_type=jax.ShapeDtypeStruct((batch, hist, DIM), jnp.float32),
        scratch_types=scratch,
    )
    def k(x_hbm, tab_hbm, scl_hbm, out_hbm, idx_v, *bufs):
        rows = bufs[0:NBUF]
        scl = bufs[NBUF:2 * NBUF]
        ob = bufs[2 * NBUF:2 * NBUF + OBUF]
        gsem = bufs[2 * NBUF + OBUF:3 * NBUF + OBUF]
        ssem = bufs[3 * NBUF + OBUF:4 * NBUF + OBUF]
        osem = bufs[4 * NBUF + OBUF:4 * NBUF + 2 * OBUF]

        wid = lax.axis_index("s") * NC + lax.axis_index("c")
        row0 = wid * rows_per_w

        pltpu.sync_copy(x_hbm.at[wid], idx_v)

        def issue_gather(c, b):
            pltpu.async_copy(tab_hbm.at[idx_v.at[c]], rows[b], gsem[b])
            pltpu.async_copy(scl_hbm.at[idx_v.at[c]], scl[b], ssem[b])

        def wait_gather(c, b):
            pltpu.make_async_copy(tab_hbm.at[idx_v.at[c]], rows[b], gsem[b]).wait()
            pltpu.make_async_copy(scl_hbm.at[idx_v.at[c]], scl[b], ssem[b]).wait()

        def issue_out(c, o):
            pltpu.async_copy(
                ob[o], out_hbm.at[pl.ds(row0 + c * BROWS, BROWS)], osem[o])

        def wait_out(o):
            pltpu.make_async_copy(
                ob[o], out_hbm.at[pl.ds(row0, BROWS)], osem[o]).wait()

        riota = lax.iota(jnp.int32, L)
        hist_i = jnp.int32(hist)

        def compute(b, o):
            rows_b, scl_b, ob_o = rows[b], scl[b], ob[o]

            def grp(g, carry):
                rb = g * L
                r = riota + rb
                d = r // hist_i
                m = r - d * hist_i
                s = plsc.load_gather(scl_b, [r])
                for j in range(WORDS):
                    jv = jnp.full((L,), j, jnp.int32)
                    w = plsc.load_gather(rows_b, [r, jv])
                    for kk in range(4):
                        col = 4 * j + kk
                        if kk < 3:
                            v = jnp.right_shift(jnp.left_shift(w, 24 - 8 * kk), 24)
                        else:
                            v = jnp.right_shift(w, 24)
                        f = v.astype(jnp.float32) * s
                        cv = jnp.full((L,), col, jnp.int32)
                        plsc.store_scatter(ob_o, [d, m, cv], f)
                return carry
            lax.fori_loop(0, n_grp, grp, 0)

        for b in range(NBUF):
            issue_gather(b, b)

        def body(i, carry):
            for b in range(NBUF):
                c = i * NBUF + b
                o = b % OBUF
                wait_gather(c, b)

                if b >= OBUF:
                    wait_out(o)
                else:
                    @pl.when(i > 0)
                    def _wo():
                        wait_out(o)

                compute(b, o)
                issue_out(c, o)

                @pl.when(i < n_outer - 1)
                def _pg():
                    issue_gather(c + NBUF, b)

            return carry

        lax.fori_loop(0, n_outer, body, 0)
        for o in range(OBUF):
            wait_out(o)

    return k(x3d, table, scales)


def kernel(x, weight_q, scales):
    batch, hist = x.shape
    num_emb = weight_q.shape[0]
    cr = BROWS * hist
    x3d = x.reshape(NW, -1, cr)
    table = _sc_prep(weight_q.reshape(num_emb // 2, 2 * DIM), num_emb)
    return _sc_lookup(x3d, table, scales, batch, hist)
